# baseline (device time: 178698 ns/iter reference)
import jax
import jax.numpy as jnp
from jax import lax
from jax.experimental import pallas as pl
from jax.experimental.pallas import tpu as pltpu

P = 384


def kernel(x, assign, W1, W2):
    t, d = x.shape
    e_loc, _, f = W1.shape
    n_exp = 2 * e_loc

    px = lax.axis_index("x")

    eids = jnp.arange(n_exp, dtype=assign.dtype)
    order = jnp.array([[0, 1, 2, 3], [2, 3, 0, 1]], jnp.int32)[px]
    keys = assign[None, :] != eids[order][:, None]
    idx = jnp.argsort(keys, axis=1, stable=True)[:, :P]
    valid = jnp.take(assign, idx) == eids[order][:, None]
    xg = (x[idx] * valid[..., None]).astype(jnp.bfloat16)

    def body(xg_ref, w1_ref, w2_ref, out_ref,
             xrecv, ysend, yrecv, w1b, w2b, send_sems, recv_sems):
        my_x = lax.axis_index("x")
        peer = (1 - my_x, lax.axis_index("y"), lax.axis_index("z"))

        barrier_sem = pltpu.get_barrier_semaphore()
        pl.semaphore_signal(barrier_sem, inc=1, device_id=peer,
                            device_id_type=pl.DeviceIdType.MESH)
        pl.semaphore_wait(barrier_sem, 1)

        rdma_x = pltpu.make_async_remote_copy(
            src_ref=xg_ref.at[pl.ds(e_loc, e_loc)], dst_ref=xrecv,
            send_sem=send_sems.at[0], recv_sem=recv_sems.at[0],
            device_id=peer, device_id_type=pl.DeviceIdType.MESH)
        rdma_x.start()

        w1b[...] = w1_ref[...].astype(jnp.bfloat16)
        w2b[...] = w2_ref[...].astype(jnp.bfloat16)

        def ffn(xb, le):
            h = jnp.dot(xb, w1b[le], preferred_element_type=jnp.float32)
            hb = jnp.maximum(h, 0.0).astype(jnp.bfloat16)
            return jnp.dot(hb, w2b[le], preferred_element_type=jnp.float32)

        for le in range(e_loc):
            out_ref[le, :, :] = ffn(xg_ref[le, :, :], le)

        rdma_x.wait()

        rdma_y = []
        for le in range(e_loc):
            ysend[le, :, :] = ffn(xrecv[le, :, :], le).astype(jnp.bfloat16)
            r = pltpu.make_async_remote_copy(
                src_ref=ysend.at[le], dst_ref=yrecv.at[le],
                send_sem=send_sems.at[1 + le], recv_sem=recv_sems.at[1 + le],
                device_id=peer, device_id_type=pl.DeviceIdType.MESH)
            r.start()
            rdma_y.append(r)

        for le in range(e_loc):
            rdma_y[le].wait()
            out_ref[e_loc + le, :, :] = yrecv[le, :, :].astype(jnp.float32)

    yall = pl.pallas_call(
        body,
        out_shape=jax.ShapeDtypeStruct((n_exp, P, d), jnp.float32),
        in_specs=[
            pl.BlockSpec(memory_space=pltpu.VMEM),
            pl.BlockSpec(memory_space=pltpu.VMEM),
            pl.BlockSpec(memory_space=pltpu.VMEM),
        ],
        out_specs=pl.BlockSpec(memory_space=pltpu.VMEM),
        scratch_shapes=[
            pltpu.VMEM((e_loc, P, d), jnp.bfloat16),
            pltpu.VMEM((e_loc, P, d), jnp.bfloat16),
            pltpu.VMEM((e_loc, P, d), jnp.bfloat16),
            pltpu.VMEM((e_loc, W1.shape[1], f), jnp.bfloat16),
            pltpu.VMEM((e_loc, f, W2.shape[2]), jnp.bfloat16),
            pltpu.SemaphoreType.DMA((3,)),
            pltpu.SemaphoreType.DMA((3,)),
        ],
        compiler_params=pltpu.CompilerParams(
            collective_id=0,
            vmem_limit_bytes=100 * 1024 * 1024,
        ),
    )(xg, W1, W2)

    out = jnp.zeros((t, d), jnp.float32)
    return out.at[idx.reshape(-1)].add(yall.reshape(n_exp * P, d))


# device time: 73794 ns/iter; 2.4216x vs baseline; 2.4216x over previous
import jax
import jax.numpy as jnp
from jax import lax
from jax.experimental import pallas as pl
from jax.experimental.pallas import tpu as pltpu

P = 384


def kernel(x, assign, W1, W2):
    t, d = x.shape
    e_loc, _, f = W1.shape
    n_exp = 2 * e_loc
    bp = e_loc * P

    px = lax.axis_index("x")

    eids = jnp.arange(n_exp, dtype=assign.dtype)
    oh = (assign[None, :] == eids[:, None]).astype(jnp.int32)
    rank = jnp.cumsum(oh, axis=1) - 1
    slot_tbl = jnp.array([[0, 1, 2, 3], [2, 3, 0, 1]], jnp.int32)[px]
    slot = jnp.sum(oh * slot_tbl[:, None], axis=0)
    dest = slot * P + jnp.sum(oh * rank, axis=0)
    dest = dest.astype(jnp.int32)
    dest_row = dest.reshape(1, t)
    dest_col = dest.reshape(t, 1)

    def body(x_ref, drow_ref, dcol_ref, w1_ref, w2_ref, out_ref,
             xg, xrecv, yl, ysend, yrecv, send_sems, recv_sems):
        peer = (1 - lax.axis_index("x"), lax.axis_index("y"),
                lax.axis_index("z"))

        barrier_sem = pltpu.get_barrier_semaphore()
        pl.semaphore_signal(barrier_sem, inc=1, device_id=peer,
                            device_id_type=pl.DeviceIdType.MESH)
        pl.semaphore_wait(barrier_sem, 1)

        rows = lax.broadcasted_iota(jnp.int32, (n_exp * P, t), 0)
        S = (rows == drow_ref[...]).astype(jnp.bfloat16)
        xb = x_ref[...].astype(jnp.bfloat16)
        xg[...] = jnp.dot(
            S, xb, preferred_element_type=jnp.float32).astype(jnp.bfloat16)

        rdma_x = pltpu.make_async_remote_copy(
            src_ref=xg.at[pl.ds(bp, bp)], dst_ref=xrecv,
            send_sem=send_sems.at[0], recv_sem=recv_sems.at[0],
            device_id=peer, device_id_type=pl.DeviceIdType.MESH)
        rdma_x.start()

        def ffn(xv, le):
            w1b = w1_ref[le].astype(jnp.bfloat16)
            w2b = w2_ref[le].astype(jnp.bfloat16)
            h = jnp.dot(xv, w1b, preferred_element_type=jnp.float32)
            hb = jnp.maximum(h, 0.0).astype(jnp.bfloat16)
            return jnp.dot(hb, w2b, preferred_element_type=jnp.float32)

        for le in range(e_loc):
            yl[le * P:(le + 1) * P, :] = ffn(
                xg[le * P:(le + 1) * P, :], le).astype(jnp.bfloat16)

        rdma_x.wait()

        rdma_y = []
        for le in range(e_loc):
            ysend[le * P:(le + 1) * P, :] = ffn(
                xrecv[le * P:(le + 1) * P, :], le).astype(jnp.bfloat16)
            r = pltpu.make_async_remote_copy(
                src_ref=ysend.at[pl.ds(le * P, P)],
                dst_ref=yrecv.at[pl.ds(le * P, P)],
                send_sem=send_sems.at[1 + le], recv_sem=recv_sems.at[1 + le],
                device_id=peer, device_id_type=pl.DeviceIdType.MESH)
            r.start()
            rdma_y.append(r)

        cols_a = lax.broadcasted_iota(jnp.int32, (t, bp), 1)
        StA = (dcol_ref[...] == cols_a).astype(jnp.bfloat16)
        StB = (dcol_ref[...] == cols_a + bp).astype(jnp.bfloat16)
        acc = jnp.dot(StA, yl[...], preferred_element_type=jnp.float32)
        for r in rdma_y:
            r.wait()
        out_ref[...] = acc + jnp.dot(StB, yrecv[...],
                                     preferred_element_type=jnp.float32)

    out = pl.pallas_call(
        body,
        out_shape=jax.ShapeDtypeStruct((t, d), jnp.float32),
        in_specs=[pl.BlockSpec(memory_space=pltpu.VMEM)] * 5,
        out_specs=pl.BlockSpec(memory_space=pltpu.VMEM),
        scratch_shapes=[
            pltpu.VMEM((n_exp * P, d), jnp.bfloat16),
            pltpu.VMEM((bp, d), jnp.bfloat16),
            pltpu.VMEM((bp, d), jnp.bfloat16),
            pltpu.VMEM((bp, d), jnp.bfloat16),
            pltpu.VMEM((bp, d), jnp.bfloat16),
            pltpu.SemaphoreType.DMA((3,)),
            pltpu.SemaphoreType.DMA((3,)),
        ],
        compiler_params=pltpu.CompilerParams(
            collective_id=0,
            vmem_limit_bytes=100 * 1024 * 1024,
        ),
    )(x, dest_row, dest_col, W1, W2)
    return out
